# baseline (speedup 1.0000x reference)
import jax
import jax.numpy as jnp
from jax import lax
from jax.experimental import pallas as pl
from jax.experimental.pallas import tpu as pltpu

_B, _N, _INF, _OUTF = 4, 1024, 256, 256
_PB = 2


def _gat_body(inp_ref, adj_ref, w_ref, a_ref, out_ref, at_s, ones_s):
    @pl.when(pl.program_id(0) == 0)
    def _():
        at_s[...] = jnp.transpose(a_ref[...])
        ones_s[...] = jnp.ones((_OUTF, 8), jnp.float32)

    m = (adj_ref[...] > 0).astype(jnp.bfloat16)
    at = at_s[...]
    ones = ones_s[...][:, :1]
    row = lax.broadcasted_iota(jnp.int32, (_N, 1), 0)
    for u in range(_PB):
        x = inp_ref[u]
        h = jnp.dot(x, w_ref[...], preferred_element_type=jnp.float32)
        h = jnp.where(row == 0, 0.0, h)
        hp = pltpu.roll(h, 1, 0)
        hp = jnp.where(row == 0, 0.0, hp).astype(jnp.bfloat16)
        g = lax.dot_general(m, hp, (((0,), (0,)), ((), ())),
                            preferred_element_type=jnp.float32)
        prod = h * at[:, :_OUTF] + g * at[:, _OUTF:]
        vals = jnp.dot(prod, ones, preferred_element_type=jnp.float32)
        vals = jnp.where(row == 0, 0.0, vals)
        ssum = jnp.sum(h, axis=0, keepdims=True)
        o = lax.dot_general(vals, ssum, (((1,), (0,)), ((), ())),
                            preferred_element_type=jnp.float32)
        out_ref[u] = jnp.maximum(o, 0.2 * o)


def kernel(inp, adj, W, a):
    return pl.pallas_call(
        _gat_body,
        grid=(_B // _PB,),
        in_specs=[
            pl.BlockSpec((_PB, _N, _INF), lambda b: (b, 0, 0)),
            pl.BlockSpec((_N, _N), lambda b: (0, 0)),
            pl.BlockSpec((_INF, _OUTF), lambda b: (0, 0)),
            pl.BlockSpec((2 * _OUTF, _N), lambda b: (0, 0)),
        ],
        out_specs=pl.BlockSpec((_PB, _N, _OUTF), lambda b: (b, 0, 0)),
        out_shape=jax.ShapeDtypeStruct((_B, _N, _OUTF), jnp.float32),
        scratch_shapes=[pltpu.VMEM((_N, 2 * _OUTF), jnp.float32),
                        pltpu.VMEM((_OUTF, 8), jnp.float32)],
        compiler_params=pltpu.CompilerParams(
            dimension_semantics=("arbitrary",),
        ),
    )(inp, adj, W, a)


# speedup vs baseline: 1.3705x; 1.3705x over previous
import jax
import jax.numpy as jnp
from jax import lax
from jax.experimental import pallas as pl
from jax.experimental.pallas import tpu as pltpu

_B, _N, _INF, _OUTF = 4, 1024, 256, 256
_PB = 2


def _gat_body(inp_ref, adj_ref, w_ref, a_ref, out_ref, at_s):
    @pl.when(pl.program_id(0) == 0)
    def _():
        at_s[...] = jnp.transpose(a_ref[...])

    m = (adj_ref[...] > 0).astype(jnp.bfloat16)
    at = at_s[...]
    row = lax.broadcasted_iota(jnp.int32, (_N, 1), 0)
    for u in range(_PB):
        x = inp_ref[u]
        h = jnp.dot(x, w_ref[...], preferred_element_type=jnp.float32)
        h = jnp.where(row == 0, 0.0, h)
        hp = pltpu.roll(h, 1, 0)
        hp = jnp.where(row == 0, 0.0, hp).astype(jnp.bfloat16)
        g = lax.dot_general(m, hp, (((0,), (0,)), ((), ())),
                            preferred_element_type=jnp.float32)
        vals = (jnp.sum(h * at[:, :_OUTF], axis=1, keepdims=True)
                + jnp.sum(g * at[:, _OUTF:], axis=1, keepdims=True))
        vals = jnp.where(row == 0, 0.0, vals)
        ssum = jnp.sum(h, axis=0, keepdims=True)
        o = vals * ssum
        out_ref[u] = jnp.maximum(o, 0.2 * o)


def kernel(inp, adj, W, a):
    return pl.pallas_call(
        _gat_body,
        grid=(_B // _PB,),
        in_specs=[
            pl.BlockSpec((_PB, _N, _INF), lambda b: (b, 0, 0)),
            pl.BlockSpec((_N, _N), lambda b: (0, 0)),
            pl.BlockSpec((_INF, _OUTF), lambda b: (0, 0)),
            pl.BlockSpec((2 * _OUTF, _N), lambda b: (0, 0)),
        ],
        out_specs=pl.BlockSpec((_PB, _N, _OUTF), lambda b: (b, 0, 0)),
        out_shape=jax.ShapeDtypeStruct((_B, _N, _OUTF), jnp.float32),
        scratch_shapes=[pltpu.VMEM((_N, 2 * _OUTF), jnp.float32)],
        compiler_params=pltpu.CompilerParams(
            dimension_semantics=("arbitrary",),
        ),
    )(inp, adj, W, a)
